# Initial kernel scaffold; baseline (speedup 1.0000x reference)
import jax
import jax.numpy as jnp
from jax.experimental import pallas as pl


def kernel(indices, C):
    raise NotImplementedError("write your pallas kernel here")


# speedup vs baseline: 5.4513x; 5.4513x over previous
import functools

import jax
import jax.numpy as jnp
from jax import lax
from jax.experimental import pallas as pl
from jax.experimental.pallas import tpu as pltpu
from jax.experimental.pallas import tpu_sc as plsc

B, S = 16384, 200
N = B * S
NC, NS, LANES = 2, 16, 16
NW = NC * NS
PER_W = N // NW
CHUNK = 12800
NCHUNK = PER_W // CHUNK
STEPS = CHUNK // LANES

_mesh = plsc.VectorSubcoreMesh(core_axis_name="c", subcore_axis_name="s")


@functools.partial(
    pl.kernel,
    out_type=jax.ShapeDtypeStruct((2 * N,), jnp.float32),
    mesh=_mesh,
    scratch_types=[
        pltpu.VMEM((128,), jnp.float32),
        pltpu.VMEM((CHUNK,), jnp.int32),
        pltpu.VMEM((2 * CHUNK,), jnp.float32),
    ],
    compiler_params=pltpu.CompilerParams(needs_layout_passes=False),
)
def _emb_lookup(tab_hbm, idx_hbm, out_hbm, tab_v, idx_v, out_v):
    wid = lax.axis_index("s") * NC + lax.axis_index("c")
    base = wid * PER_W
    pltpu.sync_copy(tab_hbm, tab_v)
    lane = lax.iota(jnp.int32, LANES)

    def chunk_body(c, carry):
        off = base + c * CHUNK
        pltpu.sync_copy(idx_hbm.at[pl.ds(off, CHUNK)], idx_v)

        def step(i, carry2):
            iv = idx_v[pl.ds(i * LANES, LANES)]
            g = iv * 2
            even = plsc.load_gather(tab_v, [g])
            odd = plsc.load_gather(tab_v, [g + 1])
            o = i * (2 * LANES) + lane * 2
            plsc.store_scatter(out_v, [o], even)
            plsc.store_scatter(out_v, [o + 1], odd)
            return carry2

        lax.fori_loop(0, STEPS, step, 0)
        pltpu.sync_copy(out_v, out_hbm.at[pl.ds(2 * off, 2 * CHUNK)])
        return carry

    lax.fori_loop(0, NCHUNK, chunk_body, 0)


def kernel(indices, C):
    tab = jnp.zeros((128,), jnp.float32).at[:72].set(C.reshape(-1))
    flat = _emb_lookup(tab, indices.reshape(-1))
    return flat.reshape(B, S, 2)


# speedup vs baseline: 5.6015x; 1.0275x over previous
import functools

import jax
import jax.numpy as jnp
from jax import lax
from jax.experimental import pallas as pl
from jax.experimental.pallas import tpu as pltpu
from jax.experimental.pallas import tpu_sc as plsc

B, S = 16384, 200
N = B * S
NC, NS, LANES = 2, 16, 16
NW = NC * NS
PER_W = N // NW
CHUNK = 12800
NCHUNK = PER_W // CHUNK
STEPS = CHUNK // LANES

_mesh = plsc.VectorSubcoreMesh(core_axis_name="c", subcore_axis_name="s")

_GDN = lax.GatherDimensionNumbers(
    offset_dims=(), collapsed_slice_dims=(0,), start_index_map=(0,))


def _take16(vec, perm):
    return lax.gather(vec, perm[:, None], _GDN, (1,),
                      mode=lax.GatherScatterMode.PROMISE_IN_BOUNDS)


@functools.partial(
    pl.kernel,
    out_type=jax.ShapeDtypeStruct((2 * N,), jnp.float32),
    mesh=_mesh,
    scratch_types=[
        pltpu.VMEM((128,), jnp.float32),
        pltpu.VMEM((CHUNK,), jnp.int32),
        pltpu.VMEM((CHUNK,), jnp.int32),
        pltpu.VMEM((2 * CHUNK,), jnp.float32),
        pltpu.VMEM((2 * CHUNK,), jnp.float32),
        pltpu.SemaphoreType.DMA,
        pltpu.SemaphoreType.DMA,
        pltpu.SemaphoreType.DMA,
        pltpu.SemaphoreType.DMA,
    ],
    compiler_params=pltpu.CompilerParams(needs_layout_passes=False),
)
def _emb_lookup(tab_hbm, idx_hbm, out_hbm, tab_v,
                idx_a, idx_b, out_a, out_b, sia, sib, soa, sob):
    wid = lax.axis_index("s") * NC + lax.axis_index("c")
    base = wid * PER_W
    pltpu.sync_copy(tab_hbm, tab_v)
    lane = lax.iota(jnp.int32, LANES)
    par = lane % 2
    perm_lo = lane // 2
    perm_hi = perm_lo + 8

    idx_bufs = (idx_a, idx_b)
    out_bufs = (out_a, out_b)
    in_sems = (sia, sib)
    out_sems = (soa, sob)

    def in_copy(c):
        return pltpu.async_copy(
            idx_hbm.at[pl.ds(base + c * CHUNK, CHUNK)],
            idx_bufs[c % 2], in_sems[c % 2])

    def out_copy(c):
        return pltpu.async_copy(
            out_bufs[c % 2],
            out_hbm.at[pl.ds(2 * (base + c * CHUNK), 2 * CHUNK)],
            out_sems[c % 2])

    pending_in = in_copy(0)
    pending_out = [None, None]
    for c in range(NCHUNK):
        nxt = in_copy(c + 1) if c + 1 < NCHUNK else None
        pending_in.wait()
        pending_in = nxt
        if pending_out[c % 2] is not None:
            pending_out[c % 2].wait()
        iv_ref, ov_ref = idx_bufs[c % 2], out_bufs[c % 2]

        @plsc.parallel_loop(0, STEPS, step=1, unroll=8)
        def step(i):
            iv = iv_ref[pl.ds(i * LANES, LANES)]
            lo = _take16(iv, perm_lo)
            hi = _take16(iv, perm_hi)
            vlo = plsc.load_gather(tab_v, [lo * 2 + par])
            vhi = plsc.load_gather(tab_v, [hi * 2 + par])
            o = i * (2 * LANES)
            ov_ref[pl.ds(o, LANES)] = vlo
            ov_ref[pl.ds(o + LANES, LANES)] = vhi

        pending_out[c % 2] = out_copy(c)
    pending_out[0].wait()
    pending_out[1].wait()


def kernel(indices, C):
    tab = jnp.zeros((128,), jnp.float32).at[:72].set(C.reshape(-1))
    flat = _emb_lookup(tab, indices.reshape(-1))
    return flat.reshape(B, S, 2)


# speedup vs baseline: 269.3661x; 48.0886x over previous
import functools

import jax
import jax.numpy as jnp
from jax import lax
from jax.experimental import pallas as pl
from jax.experimental.pallas import tpu as pltpu
from jax.experimental.pallas import tpu_sc as plsc

B, S = 16384, 200
NC, NS, LANES = 2, 16, 16
NW = NC * NS
NUNITS = S * 4
PER_W = NUNITS // NW
ROWS = 32
CSTEPS = 128 // LANES

_mesh = plsc.VectorSubcoreMesh(core_axis_name="c", subcore_axis_name="s")


@functools.partial(
    pl.kernel,
    out_type=jax.ShapeDtypeStruct((S, 128, 2, 128), jnp.float32),
    mesh=_mesh,
    scratch_types=[
        pltpu.VMEM((128,), jnp.float32),
        pltpu.VMEM((ROWS, 128), jnp.int32),
        pltpu.VMEM((ROWS, 128), jnp.int32),
        pltpu.VMEM((ROWS, 2, 128), jnp.float32),
        pltpu.VMEM((ROWS, 2, 128), jnp.float32),
        pltpu.SemaphoreType.DMA,
        pltpu.SemaphoreType.DMA,
        pltpu.SemaphoreType.DMA,
        pltpu.SemaphoreType.DMA,
    ],
    compiler_params=pltpu.CompilerParams(needs_layout_passes=False),
)
def _emb_lookup(idx_hbm, tab_hbm, out_hbm, tab_v,
                idx_a, idx_b, out_a, out_b, sia, sib, soa, sob):
    wid = lax.axis_index("s") * NC + lax.axis_index("c")
    base = wid * PER_W
    pltpu.sync_copy(tab_hbm, tab_v)

    idx_bufs = (idx_a, idx_b)
    out_bufs = (out_a, out_b)
    in_sems = (sia, sib)
    out_sems = (soa, sob)

    def in_copy(t):
        u = base + t
        j, q = u // 4, u % 4
        return pltpu.async_copy(
            idx_hbm.at[j // 8, pl.ds(q * ROWS, ROWS), j % 8],
            idx_bufs[t % 2], in_sems[t % 2])

    def out_copy(t):
        u = base + t
        j, q = u // 4, u % 4
        return pltpu.async_copy(
            out_bufs[t % 2],
            out_hbm.at[j, pl.ds(q * ROWS, ROWS)],
            out_sems[t % 2])

    pending_in = in_copy(0)
    pending_out = [None, None]
    for t in range(PER_W):
        nxt = in_copy(t + 1) if t + 1 < PER_W else None
        pending_in.wait()
        pending_in = nxt
        if pending_out[t % 2] is not None:
            pending_out[t % 2].wait()
        iv_ref, ov_ref = idx_bufs[t % 2], out_bufs[t % 2]

        @plsc.parallel_loop(0, ROWS * CSTEPS, step=1, unroll=8)
        def step(i):
            r = i // CSTEPS
            c = (i % CSTEPS) * LANES
            iv = iv_ref[r, pl.ds(c, LANES)]
            g = iv * 2
            ov_ref[r, 0, pl.ds(c, LANES)] = plsc.load_gather(tab_v, [g])
            ov_ref[r, 1, pl.ds(c, LANES)] = plsc.load_gather(tab_v, [g + 1])

        pending_out[t % 2] = out_copy(t)
    pending_out[0].wait()
    pending_out[1].wait()


def kernel(indices, C):
    tab = jnp.zeros((128,), jnp.float32).at[:72].set(C.reshape(-1))
    xr = indices.T.reshape(S // 8, 8, 128, 128).transpose(0, 2, 1, 3)
    w = _emb_lookup(xr, tab)
    return w.transpose(1, 3, 0, 2).reshape(B, S, 2)


# speedup vs baseline: 271.2391x; 1.0070x over previous
import functools

import jax
import jax.numpy as jnp
from jax import lax
from jax.experimental import pallas as pl
from jax.experimental.pallas import tpu as pltpu
from jax.experimental.pallas import tpu_sc as plsc

B, S = 16384, 200
NC, NS, LANES = 2, 16, 16
NW = NC * NS
NUNITS = S * 4
PER_W = NUNITS // NW
ROWS = 32
CSTEPS = 128 // LANES

_mesh = plsc.VectorSubcoreMesh(core_axis_name="c", subcore_axis_name="s")


@functools.partial(
    pl.kernel,
    out_type=jax.ShapeDtypeStruct((S, 128, 2, 128), jnp.float32),
    mesh=_mesh,
    scratch_types=[
        pltpu.VMEM((128,), jnp.float32),
        pltpu.VMEM((ROWS, 128), jnp.int32),
        pltpu.VMEM((ROWS, 128), jnp.int32),
        pltpu.VMEM((ROWS, 2, 128), jnp.float32),
        pltpu.VMEM((ROWS, 2, 128), jnp.float32),
        pltpu.SemaphoreType.DMA,
        pltpu.SemaphoreType.DMA,
        pltpu.SemaphoreType.DMA,
        pltpu.SemaphoreType.DMA,
    ],
    compiler_params=pltpu.CompilerParams(needs_layout_passes=False),
)
def _emb_lookup(idx_hbm, tab_hbm, out_hbm, tab_v,
                idx_a, idx_b, out_a, out_b, sia, sib, soa, sob):
    wid = lax.axis_index("s") * NC + lax.axis_index("c")
    base = wid * PER_W

    idx_bufs = (idx_a, idx_b)
    out_bufs = (out_a, out_b)
    in_sems = (sia, sib)
    out_sems = (soa, sob)

    def in_copy(t):
        u = base + t
        j, q = u // 4, u % 4
        return pltpu.async_copy(
            idx_hbm.at[j // 8, pl.ds(q * ROWS, ROWS), j % 8],
            idx_bufs[t % 2], in_sems[t % 2])

    def out_copy(t):
        u = base + t
        j, q = u // 4, u % 4
        return pltpu.async_copy(
            out_bufs[t % 2],
            out_hbm.at[j, pl.ds(q * ROWS, ROWS)],
            out_sems[t % 2])

    pending_in = in_copy(0)
    pltpu.sync_copy(tab_hbm, tab_v)
    pending_out = [None, None]
    for t in range(PER_W):
        nxt = in_copy(t + 1) if t + 1 < PER_W else None
        pending_in.wait()
        pending_in = nxt
        if pending_out[t % 2] is not None:
            pending_out[t % 2].wait()
        iv_ref, ov_ref = idx_bufs[t % 2], out_bufs[t % 2]

        @plsc.parallel_loop(0, ROWS * CSTEPS, step=1, unroll=8)
        def step(i):
            r = i // CSTEPS
            c = (i % CSTEPS) * LANES
            iv = iv_ref[r, pl.ds(c, LANES)]
            g = iv * 2
            ov_ref[r, 0, pl.ds(c, LANES)] = plsc.load_gather(tab_v, [g])
            ov_ref[r, 1, pl.ds(c, LANES)] = plsc.load_gather(tab_v, [g + 1])

        pending_out[t % 2] = out_copy(t)
    pending_out[0].wait()
    pending_out[1].wait()


def kernel(indices, C):
    tab = jnp.zeros((128,), jnp.float32).at[:72].set(C.reshape(-1))
    xr = indices.T.reshape(S // 8, 8, 128, 128).transpose(0, 2, 1, 3)
    w = _emb_lookup(xr, tab)
    return w.transpose(1, 3, 0, 2).reshape(B, S, 2)
